# Initial kernel scaffold; baseline (speedup 1.0000x reference)
import jax
import jax.numpy as jnp
from jax.experimental import pallas as pl


def kernel(device_idx, channel_idx, device_cat, channel_id, device_cont, channel_cont, emb_lang, emb_plat, emb_os, emb_country, emb_carrier, emb_brand, emb_platos, emb_chan_id, W_cl1, b_cl1, W_mp1, b_mp1, W_f1, b_f1, W_d1, b_d1, W_d2, b_d2, W_c1, b_c1, W_c2, b_c2, W_c3, b_c3):
    raise NotImplementedError("write your pallas kernel here")


# speedup vs baseline: 3.2975x; 3.2975x over previous
import functools

import jax
import jax.numpy as jnp
from jax import lax
from jax.experimental import pallas as pl
from jax.experimental.pallas import tpu as pltpu
from jax.experimental.pallas import tpu_sc as plsc

N_DEV = 50000
N_CHAN = 512
E = 800000
EMB = 16
CARDS = [50, 4, 30, 200, 300, 500, 100]
PAD8 = [56, 8, 32, 200, 304, 504, 104]
OFF8 = [0, 56, 64, 96, 296, 600, 1104]
KTOT = 1280

NC, NS = 2, 16
NW = NC * NS

CE = 128
NCHE = E // CE
TE = -(-NCHE // NW)

CNT_COL = 48

_MESH = plsc.VectorSubcoreMesh(core_axis_name="c", subcore_axis_name="s")


@functools.partial(
    pl.kernel,
    out_type=jax.ShapeDtypeStruct((NC, N_CHAN, 128), jnp.float32),
    mesh=_MESH,
    scratch_types=[
        pltpu.VMEM((CE,), jnp.int32),
        pltpu.VMEM((CE,), jnp.int32),
        pltpu.VMEM((CE, 128), jnp.float32),
        pltpu.VMEM_SHARED((N_CHAN, 128), jnp.float32),
        pltpu.SemaphoreType.DMA,
    ],
)
def _sca_segsum(di_ref, ci_ref, g_ref, zeros_ref, out_ref,
                di_v, ci_v, rows_v, acc_sh, sem):
    cidx = lax.axis_index("c")
    sid = lax.axis_index("s")
    wid = sid * NC + cidx

    @pl.when(sid == 0)
    def _():
        pltpu.sync_copy(zeros_ref, acc_sh)

    plsc.subcore_barrier()

    def body(k, carry):
        c = wid + NW * k

        @pl.when(c < NCHE)
        def _():
            base = c * CE
            pltpu.sync_copy(di_ref.at[pl.ds(base, CE)], di_v)
            pltpu.sync_copy(ci_ref.at[pl.ds(base, CE)], ci_v)
            pltpu.async_copy(g_ref.at[di_v], rows_v, sem).wait()
            pltpu.sync_copy(rows_v, acc_sh.at[ci_v], add=True)
        return carry

    lax.fori_loop(0, TE, body, 0)
    plsc.subcore_barrier()

    @pl.when(sid == 0)
    def _():
        pltpu.sync_copy(acc_sh, out_ref.at[cidx])


@functools.partial(
    pl.kernel,
    out_type=jax.ShapeDtypeStruct((E // 2, 128), jnp.float32),
    mesh=_MESH,
    scratch_types=[
        pltpu.VMEM((CE,), jnp.int32),
        pltpu.VMEM((CE,), jnp.int32),
        pltpu.VMEM((CE, 128), jnp.float32),
        pltpu.VMEM((CE, 128), jnp.float32),
        pltpu.VMEM((CE // 2, 128), jnp.float32),
        pltpu.VMEM_SHARED((N_CHAN, 128), jnp.float32),
        pltpu.SemaphoreType.DMA,
        pltpu.SemaphoreType.DMA,
    ],
)
def _scb_combine(di_ref, ci_ref, g_ref, fc_ref, out_ref,
                 di_v, ci_v, r1_v, r2_v, w_v, fc_sh, sem1, sem2):
    cidx = lax.axis_index("c")
    sid = lax.axis_index("s")
    wid = sid * NC + cidx

    @pl.when(sid == 0)
    def _():
        pltpu.sync_copy(fc_ref, fc_sh)

    plsc.subcore_barrier()

    def body(k, carry):
        c = wid + NW * k

        @pl.when(c < NCHE)
        def _():
            base = c * CE
            pltpu.sync_copy(di_ref.at[pl.ds(base, CE)], di_v)
            pltpu.sync_copy(ci_ref.at[pl.ds(base, CE)], ci_v)
            cp1 = pltpu.async_copy(g_ref.at[di_v], r1_v, sem1)
            cp2 = pltpu.async_copy(fc_sh.at[ci_v], r2_v, sem2)
            cp1.wait()
            cp2.wait()

            def pack(p, cc):
                for half in range(2):
                    e = 2 * p + half
                    for cg in range(4):
                        w_v[p, pl.ds(64 * half + 16 * cg, 16)] = (
                            r1_v[e, pl.ds(64 + 16 * cg, 16)]
                            + r2_v[e, pl.ds(16 * cg, 16)])
                return cc

            lax.fori_loop(0, CE // 2, pack, 0)
            base2 = pl.multiple_of(c * (CE // 2), 8)
            pltpu.sync_copy(w_v, out_ref.at[pl.ds(base2, CE // 2)])
        return carry

    lax.fori_loop(0, TE, body, 0)


def _tc0_body(emb_ref, wmp_ref, wd1_ref, wf1b_ref, tmp2_ref, td1_ref):
    zpad = jnp.zeros((KTOT - OFF8[-1] - PAD8[-1], 128), jnp.float32)

    def fold(w_ref):
        pieces = []
        for j in range(7):
            pieces.append(jnp.dot(
                emb_ref[pl.ds(OFF8[j], PAD8[j]), :],
                w_ref[pl.ds(16 * j, 16), :],
                preferred_element_type=jnp.float32))
        pieces.append(zpad)
        return jnp.concatenate(pieces, axis=0)

    tmp = fold(wmp_ref)
    tmp2_ref[...] = jnp.dot(tmp, wf1b_ref[...],
                            preferred_element_type=jnp.float32)
    td1_ref[...] = fold(wd1_ref)


def _tc1_body(cat_ref, cont_ref, tmp2_ref, td1_ref, wmp0f_ref, wd10_ref,
              bd1_ref, wd2_ref, bd2_ref, wc1d_ref, g_ref):
    cat = cat_ref[...]
    cont = cont_ref[...]
    lanes = lax.broadcasted_iota(jnp.int32, (cat.shape[0], KTOT), 1)
    hit = (lanes == cat[:, 0:1] + OFF8[0])
    for j in range(1, 7):
        hit = hit | (lanes == cat[:, j:j + 1] + OFF8[j])
    mh = hit.astype(jnp.float32)
    p2 = jnp.dot(mh, tmp2_ref[...], preferred_element_type=jnp.float32)
    p2 = p2 + cont * wmp0f_ref[...]
    lane64 = lax.broadcasted_iota(jnp.int32, p2.shape, 1)
    p2 = jnp.where(lane64 == CNT_COL, 1.0, p2)
    h1 = jnp.dot(mh, td1_ref[...], preferred_element_type=jnp.float32)
    h1 = jnp.maximum(h1 + cont * wd10_ref[...] + bd1_ref[...], 0.0)
    h2 = jnp.maximum(
        jnp.dot(h1, wd2_ref[...], preferred_element_type=jnp.float32)
        + bd2_ref[...], 0.0)
    hc = jnp.dot(h2, wc1d_ref[...], preferred_element_type=jnp.float32)
    g_ref[:, 0:64] = p2
    g_ref[:, 64:128] = hc


def _tc2_body(part_ref, chin_ref, wcl_ref, bcl_ref, wf1a_ref, bf1t_ref,
              wc1a_ref, bc1_ref, fc_ref):
    s = part_ref[0] + part_ref[1]
    cnt = s[:, CNT_COL:CNT_COL + 1]
    inv = 1.0 / jnp.maximum(cnt, 1.0)
    hch = jnp.dot(chin_ref[...], wcl_ref[...],
                  preferred_element_type=jnp.float32) + bcl_ref[...]
    f = jnp.maximum(
        jnp.dot(hch, wf1a_ref[...], preferred_element_type=jnp.float32)
        + s[:, 0:64] * inv + bf1t_ref[...], 0.0)
    fc_ref[...] = jnp.dot(f, wc1a_ref[...],
                          preferred_element_type=jnp.float32) + bc1_ref[...]


def _tc3_body(x_ref, wc2_ref, bc2_ref, w3_ref, b3_ref, out_ref):
    s = jnp.maximum(x_ref[...], 0.0)
    h2 = jnp.maximum(
        jnp.dot(s, wc2_ref[...], preferred_element_type=jnp.float32)
        + bc2_ref[...], 0.0)
    out_ref[...] = jnp.dot(h2, w3_ref[...],
                           preferred_element_type=jnp.float32) + b3_ref[...]


def _pad2(a, r, c):
    return jnp.pad(a, ((0, r - a.shape[0]), (0, c - a.shape[1])))


def kernel(device_idx, channel_idx, device_cat, channel_id, device_cont,
           channel_cont, emb_lang, emb_plat, emb_os, emb_country,
           emb_carrier, emb_brand, emb_platos, emb_chan_id,
           W_cl1, b_cl1, W_mp1, b_mp1, W_f1, b_f1, W_d1, b_d1, W_d2, b_d2,
           W_c1, b_c1, W_c2, b_c2, W_c3, b_c3):
    f32 = jnp.float32
    di = device_idx.astype(jnp.int32)
    ci = channel_idx.astype(jnp.int32)

    tabs = [emb_lang, emb_plat, emb_os, emb_country, emb_carrier, emb_brand,
            emb_platos]
    emb_cat = jnp.concatenate(
        [_pad2(t.astype(f32), p, EMB) for t, p in zip(tabs, PAD8)],
        axis=0)
    emb_cat = _pad2(emb_cat, KTOT, EMB)
    wmp_r = _pad2(W_mp1[1:, :], 112, 128)
    wd1_r = _pad2(W_d1[1:, :], 112, 128)
    wf1b = _pad2(W_f1[14:, :], 128, 64)

    tmp2, td1 = pl.pallas_call(
        _tc0_body,
        out_shape=[
            jax.ShapeDtypeStruct((KTOT, 64), f32),
            jax.ShapeDtypeStruct((KTOT, 128), f32),
        ],
    )(emb_cat, wmp_r, wd1_r, wf1b)

    wmp0f = _pad2(W_mp1[0:1, :] @ W_f1[14:, :], 1, 64)
    wd10 = _pad2(W_d1[0:1, :], 1, 128)
    bd1 = _pad2(b_d1[None, :], 1, 128)
    wd2 = _pad2(W_d2, 128, 128)
    bd2 = _pad2(b_d2[None, :], 1, 128)
    wc1d = _pad2(W_c1[48:, :], 128, 64)
    B1 = 2000
    g1 = N_DEV // B1
    g_tab = pl.pallas_call(
        _tc1_body,
        grid=(g1,),
        in_specs=[
            pl.BlockSpec((B1, 7), lambda i: (i, 0)),
            pl.BlockSpec((B1, 1), lambda i: (i, 0)),
            pl.BlockSpec((KTOT, 64), lambda i: (0, 0)),
            pl.BlockSpec((KTOT, 128), lambda i: (0, 0)),
            pl.BlockSpec((1, 64), lambda i: (0, 0)),
            pl.BlockSpec((1, 128), lambda i: (0, 0)),
            pl.BlockSpec((1, 128), lambda i: (0, 0)),
            pl.BlockSpec((128, 128), lambda i: (0, 0)),
            pl.BlockSpec((1, 128), lambda i: (0, 0)),
            pl.BlockSpec((128, 64), lambda i: (0, 0)),
        ],
        out_specs=pl.BlockSpec((B1, 128), lambda i: (i, 0)),
        out_shape=jax.ShapeDtypeStruct((N_DEV, 128), f32),
    )(device_cat.astype(jnp.int32), device_cont.astype(f32), tmp2, td1,
      wmp0f, wd10, bd1, wd2, bd2, wc1d)

    zeros = jnp.zeros((N_CHAN, 128), f32)
    partial = _sca_segsum(di, ci, g_tab, zeros)

    chin = _pad2(jnp.concatenate(
        [channel_cont.astype(f32), emb_chan_id.astype(f32)], axis=1),
        N_CHAN, 32)
    wcl = _pad2(W_cl1, 32, 16)
    bcl = _pad2(b_cl1[None, :], 1, 16)
    wf1a = _pad2(W_f1[:14, :], 16, 64)
    bf1t = _pad2((b_f1 + b_mp1 @ W_f1[14:, :])[None, :], 1, 64)
    wc1a = _pad2(W_c1[:48, :], 64, 128)
    bc1 = _pad2(b_c1[None, :], 1, 128)
    fc_tab = pl.pallas_call(
        _tc2_body,
        out_shape=jax.ShapeDtypeStruct((N_CHAN, 128), f32),
    )(partial, chin, wcl, bcl, wf1a, bf1t, wc1a, bc1)

    packed = _scb_combine(di, ci, g_tab, fc_tab)

    wc2p = _pad2(W_c2, 64, 32)
    wc2_2 = jnp.zeros((128, 64), f32)
    wc2_2 = wc2_2.at[0:64, 0:32].set(wc2p)
    wc2_2 = wc2_2.at[64:128, 32:64].set(wc2p)
    bc2p = _pad2(b_c2[None, :], 1, 32)
    bc2_2 = jnp.concatenate([bc2p, bc2p], axis=1)
    w3p = _pad2(W_c3, 32, 1)
    w3_2 = jnp.zeros((64, 8), f32)
    w3_2 = w3_2.at[0:32, 0:1].set(w3p)
    w3_2 = w3_2.at[32:64, 1:2].set(w3p)
    b3_2 = _pad2(jnp.broadcast_to(b_c3[None, :], (1, 2)), 1, 8)
    B3 = 2000
    g3 = (E // 2) // B3
    out = pl.pallas_call(
        _tc3_body,
        grid=(g3,),
        in_specs=[
            pl.BlockSpec((B3, 128), lambda i: (i, 0)),
            pl.BlockSpec((128, 64), lambda i: (0, 0)),
            pl.BlockSpec((1, 64), lambda i: (0, 0)),
            pl.BlockSpec((64, 8), lambda i: (0, 0)),
            pl.BlockSpec((1, 8), lambda i: (0, 0)),
        ],
        out_specs=pl.BlockSpec((B3, 8), lambda i: (i, 0)),
        out_shape=jax.ShapeDtypeStruct((E // 2, 8), f32),
    )(packed, wc2_2, bc2_2, w3_2, b3_2)
    return out[:, 0:2].reshape(E)


# speedup vs baseline: 5.0048x; 1.5178x over previous
import functools

import jax
import jax.numpy as jnp
from jax import lax
from jax.experimental import pallas as pl
from jax.experimental.pallas import tpu as pltpu
from jax.experimental.pallas import tpu_sc as plsc

N_DEV = 50000
N_CHAN = 512
E = 800000
EMB = 16
CARDS = [50, 4, 30, 200, 300, 500, 100]
PAD8 = [56, 8, 32, 200, 304, 504, 104]
OFF8 = [0, 56, 64, 96, 296, 600, 1104]
KTOT = 1280

NC, NS = 2, 16
NW = NC * NS

CE = 128
NCHE = E // CE
CPW = NCHE // NW
CREM = NCHE - CPW * NW

CNT_COL = 48

_MESH = plsc.VectorSubcoreMesh(core_axis_name="c", subcore_axis_name="s")


@functools.partial(
    pl.kernel,
    out_type=(
        jax.ShapeDtypeStruct((NC, N_CHAN, 128), jnp.float32),
        jax.ShapeDtypeStruct((E // 2, 128), jnp.float32),
    ),
    mesh=_MESH,
    scratch_types=[
        pltpu.VMEM((4 * CE,), jnp.int32),
        pltpu.VMEM((4, CE), jnp.int32),
        pltpu.VMEM((4, CE, 128), jnp.float32),
        pltpu.VMEM((4, CE // 2, 128), jnp.float32),
        pltpu.VMEM_SHARED((N_CHAN, 128), jnp.float32),
        pltpu.SemaphoreType.DMA,
        pltpu.SemaphoreType.DMA,
        pltpu.SemaphoreType.DMA,
    ],
)
def _sc_edges(di_ref, ci_ref, g_ref, zeros_ref, acc_ref, out_ref,
              di_v, ci_v, rows_v, w_v, acc_sh, gsem, ssem, osem):
    cidx = lax.axis_index("c")
    sid = lax.axis_index("s")
    wid = sid * NC + cidx

    @pl.when(sid == 0)
    def _():
        pltpu.sync_copy(zeros_ref, acc_sh)

    plsc.subcore_barrier()

    start = wid * CPW + jnp.minimum(wid, CREM)
    nchunks = CPW + jnp.where(wid < CREM, 1, 0)
    nfull = nchunks // 4

    def pack_chunk(j):
        def prow(p, c):
            for half in range(2):
                for cg in range(4):
                    w_v[j, p, pl.ds(64 * half + 16 * cg, 16)] = (
                        rows_v[j, 2 * p + half, pl.ds(64 + 16 * cg, 16)])
            return c
        lax.fori_loop(0, CE // 2, prow, 0)

    def full_iter(m, carry):
        cb = start + 4 * m
        base_e = cb * CE
        pltpu.sync_copy(di_ref.at[pl.ds(base_e, 4 * CE)], di_v)
        gcps = [
            pltpu.async_copy(
                g_ref.at[di_v.at[pl.ds(j * CE, CE)]], rows_v.at[j], gsem)
            for j in range(4)
        ]
        for j in range(4):
            pltpu.sync_copy(ci_ref.at[pl.ds(base_e + j * CE, CE)],
                            ci_v.at[j])
        scps = []
        ocps = []
        for j in range(4):
            gcps[j].wait()
            scps.append(pltpu.async_copy(
                rows_v.at[j], acc_sh.at[ci_v.at[j]], ssem, add=True))
            pack_chunk(j)
            ob = pl.multiple_of((cb + j) * (CE // 2), 8)
            ocps.append(pltpu.async_copy(
                w_v.at[j], out_ref.at[pl.ds(ob, CE // 2)], osem))
        for cp in scps:
            cp.wait()
        for cp in ocps:
            cp.wait()
        return carry

    lax.fori_loop(0, nfull, full_iter, 0)

    rem = nchunks - 4 * nfull
    for j in range(3):
        @pl.when(j < rem)
        def _():
            c = start + 4 * nfull + j
            base_e = c * CE
            pltpu.sync_copy(di_ref.at[pl.ds(base_e, CE)],
                            di_v.at[pl.ds(0, CE)])
            pltpu.async_copy(
                g_ref.at[di_v.at[pl.ds(0, CE)]], rows_v.at[0], gsem).wait()
            pltpu.sync_copy(ci_ref.at[pl.ds(base_e, CE)], ci_v.at[0])
            scp = pltpu.async_copy(
                rows_v.at[0], acc_sh.at[ci_v.at[0]], ssem, add=True)
            pack_chunk(0)
            ob = pl.multiple_of(c * (CE // 2), 8)
            pltpu.async_copy(
                w_v.at[0], out_ref.at[pl.ds(ob, CE // 2)], osem).wait()
            scp.wait()

    plsc.subcore_barrier()

    @pl.when(sid == 0)
    def _():
        pltpu.sync_copy(acc_sh, acc_ref.at[cidx])


def _tc0_body(emb_ref, wmp_ref, wd1_ref, wf1b_ref, tmp2_ref, td1_ref):
    zpad = jnp.zeros((KTOT - OFF8[-1] - PAD8[-1], 128), jnp.float32)

    def fold(w_ref):
        pieces = []
        for j in range(7):
            pieces.append(jnp.dot(
                emb_ref[pl.ds(OFF8[j], PAD8[j]), :],
                w_ref[pl.ds(16 * j, 16), :],
                preferred_element_type=jnp.float32))
        pieces.append(zpad)
        return jnp.concatenate(pieces, axis=0)

    tmp = fold(wmp_ref)
    tmp2_ref[...] = jnp.dot(tmp, wf1b_ref[...],
                            preferred_element_type=jnp.float32)
    td1_ref[...] = fold(wd1_ref)


def _tc1_body(cat_ref, cont_ref, tmp2_ref, td1_ref, wmp0f_ref, wd10_ref,
              bd1_ref, wd2_ref, bd2_ref, wc1d_ref, g_ref):
    cat = cat_ref[...]
    cont = cont_ref[...]
    lanes = lax.broadcasted_iota(jnp.int32, (cat.shape[0], KTOT), 1)
    hit = (lanes == cat[:, 0:1] + OFF8[0])
    for j in range(1, 7):
        hit = hit | (lanes == cat[:, j:j + 1] + OFF8[j])
    mh = hit.astype(jnp.float32)
    p2 = jnp.dot(mh, tmp2_ref[...], preferred_element_type=jnp.float32)
    p2 = p2 + cont * wmp0f_ref[...]
    lane64 = lax.broadcasted_iota(jnp.int32, p2.shape, 1)
    p2 = jnp.where(lane64 == CNT_COL, 1.0, p2)
    h1 = jnp.dot(mh, td1_ref[...], preferred_element_type=jnp.float32)
    h1 = jnp.maximum(h1 + cont * wd10_ref[...] + bd1_ref[...], 0.0)
    h2 = jnp.maximum(
        jnp.dot(h1, wd2_ref[...], preferred_element_type=jnp.float32)
        + bd2_ref[...], 0.0)
    hc = jnp.dot(h2, wc1d_ref[...], preferred_element_type=jnp.float32)
    g_ref[:, 0:64] = p2
    g_ref[:, 64:128] = hc


def _tc2_body(part_ref, chin_ref, wcl_ref, bcl_ref, wf1a_ref, bf1t_ref,
              wc1a_ref, bc1_ref, fc_ref):
    s = part_ref[0] + part_ref[1]
    cnt = s[:, CNT_COL:CNT_COL + 1]
    inv = 1.0 / jnp.maximum(cnt, 1.0)
    hch = jnp.dot(chin_ref[...], wcl_ref[...],
                  preferred_element_type=jnp.float32) + bcl_ref[...]
    f = jnp.maximum(
        jnp.dot(hch, wf1a_ref[...], preferred_element_type=jnp.float32)
        + s[:, 0:64] * inv + bf1t_ref[...], 0.0)
    fc_ref[...] = jnp.dot(f, wc1a_ref[...],
                          preferred_element_type=jnp.float32) + bc1_ref[...]


def _tc3_body(x_ref, cie_ref, fc_ref, wc2_ref, bc2_ref, w3a_ref, w3b_ref,
              b3_ref, out_ref):
    x = x_ref[...]
    cie = cie_ref[...]
    lanes = lax.broadcasted_iota(
        jnp.int32, (x.shape[0], N_CHAN), 1).astype(jnp.float32)
    oe = jnp.where(lanes == cie[:, 0:1], 1.0, 0.0)
    oo = jnp.where(lanes == cie[:, 1:2], 1.0, 0.0)
    fc = fc_ref[...]
    fce = jnp.dot(oe, fc, preferred_element_type=jnp.float32)
    fco = jnp.dot(oo, fc, preferred_element_type=jnp.float32)
    se = jnp.maximum(x[:, 0:64] + fce, 0.0)
    so = jnp.maximum(x[:, 64:128] + fco, 0.0)
    wc2 = wc2_ref[...]
    bc2 = bc2_ref[...]
    h2e = jnp.maximum(
        jnp.dot(se, wc2, preferred_element_type=jnp.float32) + bc2, 0.0)
    h2o = jnp.maximum(
        jnp.dot(so, wc2, preferred_element_type=jnp.float32) + bc2, 0.0)
    out_ref[...] = (
        jnp.dot(h2e, w3a_ref[...], preferred_element_type=jnp.float32)
        + jnp.dot(h2o, w3b_ref[...], preferred_element_type=jnp.float32)
        + b3_ref[...])


def _pad2(a, r, c):
    return jnp.pad(a, ((0, r - a.shape[0]), (0, c - a.shape[1])))


def kernel(device_idx, channel_idx, device_cat, channel_id, device_cont,
           channel_cont, emb_lang, emb_plat, emb_os, emb_country,
           emb_carrier, emb_brand, emb_platos, emb_chan_id,
           W_cl1, b_cl1, W_mp1, b_mp1, W_f1, b_f1, W_d1, b_d1, W_d2, b_d2,
           W_c1, b_c1, W_c2, b_c2, W_c3, b_c3):
    f32 = jnp.float32
    di = device_idx.astype(jnp.int32)
    ci = channel_idx.astype(jnp.int32)

    tabs = [emb_lang, emb_plat, emb_os, emb_country, emb_carrier, emb_brand,
            emb_platos]
    emb_cat = jnp.concatenate(
        [_pad2(t.astype(f32), p, EMB) for t, p in zip(tabs, PAD8)],
        axis=0)
    emb_cat = _pad2(emb_cat, KTOT, EMB)
    wmp_r = _pad2(W_mp1[1:, :], 112, 128)
    wd1_r = _pad2(W_d1[1:, :], 112, 128)
    wf1b = _pad2(W_f1[14:, :], 128, 64)

    tmp2, td1 = pl.pallas_call(
        _tc0_body,
        out_shape=[
            jax.ShapeDtypeStruct((KTOT, 64), f32),
            jax.ShapeDtypeStruct((KTOT, 128), f32),
        ],
    )(emb_cat, wmp_r, wd1_r, wf1b)

    wmp0f = _pad2(W_mp1[0:1, :] @ W_f1[14:, :], 1, 64)
    wd10 = _pad2(W_d1[0:1, :], 1, 128)
    bd1 = _pad2(b_d1[None, :], 1, 128)
    wd2 = _pad2(W_d2, 128, 128)
    bd2 = _pad2(b_d2[None, :], 1, 128)
    wc1d = _pad2(W_c1[48:, :], 128, 64)
    B1 = 2000
    g1 = N_DEV // B1
    g_tab = pl.pallas_call(
        _tc1_body,
        grid=(g1,),
        in_specs=[
            pl.BlockSpec((B1, 7), lambda i: (i, 0)),
            pl.BlockSpec((B1, 1), lambda i: (i, 0)),
            pl.BlockSpec((KTOT, 64), lambda i: (0, 0)),
            pl.BlockSpec((KTOT, 128), lambda i: (0, 0)),
            pl.BlockSpec((1, 64), lambda i: (0, 0)),
            pl.BlockSpec((1, 128), lambda i: (0, 0)),
            pl.BlockSpec((1, 128), lambda i: (0, 0)),
            pl.BlockSpec((128, 128), lambda i: (0, 0)),
            pl.BlockSpec((1, 128), lambda i: (0, 0)),
            pl.BlockSpec((128, 64), lambda i: (0, 0)),
        ],
        out_specs=pl.BlockSpec((B1, 128), lambda i: (i, 0)),
        out_shape=jax.ShapeDtypeStruct((N_DEV, 128), f32),
    )(device_cat.astype(jnp.int32), device_cont.astype(f32), tmp2, td1,
      wmp0f, wd10, bd1, wd2, bd2, wc1d)

    zeros = jnp.zeros((N_CHAN, 128), f32)
    partial, packed = _sc_edges(di, ci, g_tab, zeros)

    chin = _pad2(jnp.concatenate(
        [channel_cont.astype(f32), emb_chan_id.astype(f32)], axis=1),
        N_CHAN, 32)
    wcl = _pad2(W_cl1, 32, 16)
    bcl = _pad2(b_cl1[None, :], 1, 16)
    wf1a = _pad2(W_f1[:14, :], 16, 64)
    bf1t = _pad2((b_f1 + b_mp1 @ W_f1[14:, :])[None, :], 1, 64)
    wc1a = _pad2(W_c1[:48, :], 64, 64)
    bc1 = _pad2(b_c1[None, :], 1, 64)
    fc_tab = pl.pallas_call(
        _tc2_body,
        out_shape=jax.ShapeDtypeStruct((N_CHAN, 64), f32),
    )(partial, chin, wcl, bcl, wf1a, bf1t, wc1a, bc1)

    cie = ci.astype(f32).reshape(E // 2, 2)
    wc2p = _pad2(W_c2, 64, 32)
    bc2p = _pad2(b_c2[None, :], 1, 32)
    w3p = _pad2(W_c3, 32, 8)
    w3a = w3p
    w3b = jnp.zeros((32, 8), f32).at[:, 1:2].set(w3p[:, 0:1])
    b3 = _pad2(jnp.broadcast_to(b_c3[None, :], (1, 2)), 1, 8)
    B3 = 2000
    g3 = (E // 2) // B3
    out = pl.pallas_call(
        _tc3_body,
        grid=(g3,),
        in_specs=[
            pl.BlockSpec((B3, 128), lambda i: (i, 0)),
            pl.BlockSpec((B3, 2), lambda i: (i, 0)),
            pl.BlockSpec((N_CHAN, 64), lambda i: (0, 0)),
            pl.BlockSpec((64, 32), lambda i: (0, 0)),
            pl.BlockSpec((1, 32), lambda i: (0, 0)),
            pl.BlockSpec((32, 8), lambda i: (0, 0)),
            pl.BlockSpec((32, 8), lambda i: (0, 0)),
            pl.BlockSpec((1, 8), lambda i: (0, 0)),
        ],
        out_specs=pl.BlockSpec((B3, 8), lambda i: (i, 0)),
        out_shape=jax.ShapeDtypeStruct((E // 2, 8), f32),
    )(packed, cie, fc_tab, wc2p, bc2p, w3a, w3b, b3)
    return out[:, 0:2].reshape(E)
